# baseline (speedup 1.0000x reference)
import functools

import jax
import jax.numpy as jnp
from jax import lax
from jax.experimental import pallas as pl
from jax.experimental.pallas import tpu as pltpu
from jax.experimental.pallas import tpu_sc as plsc

_TOP_N = 16
_NT = 1024
_L = 16
_NC, _NS = 2, 16
_NW = _NC * _NS


def _tc_body(q_ref, ck_ref, scores_ref, imp_ref):
    nh = q_ref.shape[1]
    c = ck_ref.shape[2]
    d = q_ref.shape[3]
    scale = d ** -0.5
    eye = jnp.eye(c, dtype=jnp.float32)
    for h in range(nh):
        qh = q_ref[0, h]
        ckh = ck_ref[0, h]
        s = jnp.dot(qh, ckh.T, preferred_element_type=jnp.float32) * scale
        m = jnp.max(s, axis=-1, keepdims=True)
        e = jnp.exp(s - m)
        p = e / jnp.sum(e, axis=-1, keepdims=True)
        scores_ref[0, h] = jax.lax.dot_general(
            eye, p, (((1,), (1,)), ((), ())),
            precision=jax.lax.Precision.HIGHEST,
            preferred_element_type=jnp.float32)
        if h == 0:
            imp_ref[0] = p
        else:
            imp_ref[0] += p


def _tc_body_alias(q_ref, ck_ref, prev_ref, scores_ref, imp_ref):
    del prev_ref
    _tc_body(q_ref, ck_ref, scores_ref, imp_ref)


def _scores_importance_b(q, compressed_k, b, scores_prev):
    B, H, N, D = q.shape
    C = compressed_k.shape[2]
    grid = (N // _NT,)
    out_specs = [
        pl.BlockSpec((1, H, _NT, C), lambda n: (b, 0, n, 0)),
        pl.BlockSpec((1, _NT, C), lambda n: (0, n, 0)),
    ]
    out_shape = [
        jax.ShapeDtypeStruct((B, H, N, C), jnp.float32),
        jax.ShapeDtypeStruct((1, N, C), jnp.float32),
    ]
    in_specs = [
        pl.BlockSpec((1, H, _NT, D), lambda n: (b, 0, n, 0)),
        pl.BlockSpec((1, H, C, D), lambda n: (b, 0, 0, 0)),
    ]
    if scores_prev is None:
        return pl.pallas_call(
            _tc_body,
            grid=grid,
            in_specs=in_specs,
            out_specs=out_specs,
            out_shape=out_shape,
        )(q, compressed_k)
    return pl.pallas_call(
        _tc_body_alias,
        grid=grid,
        in_specs=in_specs + [pl.BlockSpec(memory_space=pl.ANY)],
        out_specs=out_specs,
        out_shape=out_shape,
        input_output_aliases={2: 0},
    )(q, compressed_k, scores_prev)


def _lane_gather(x, idx):
    return x.at[idx].get(mode="promise_in_bounds")


def _tie_fix(k, i):
    lane = lax.iota(jnp.int32, _L)
    up = jnp.minimum(lane + 1, _L - 1)
    dn = jnp.maximum(lane - 1, 0)
    kn = _lane_gather(k, up)
    for parity in (0, 1):
        inn = _lane_gather(i, up)
        swap = (k == kn) & (i > inn) & ((lane & 1) == parity)
        sw = swap.astype(jnp.int32)
        swd = _lane_gather(sw, dn) > 0
        idn = _lane_gather(i, dn)
        i = jnp.where(swap, inn, jnp.where(swd, idn, i))
    return i


def _sorted_desc(k, i):
    k2, i2 = plsc.sort_key_val(k, i, descending=True)
    return k2, _tie_fix(k2, i2)


def _merge_top(ka, ia, kb, ib):
    kbr = lax.rev(kb, (0,))
    ibr = lax.rev(ib, (0,))
    ta = (ka > kbr) | ((ka == kbr) & (ia < ibr))
    return jnp.where(ta, ka, kbr), jnp.where(ta, ia, ibr)


def _row_topk(imp_v, out_v, r):
    ks, js = [], []
    for g in range(4):
        kk = imp_v[r, pl.ds(g * _L, _L)]
        ii = lax.iota(jnp.int32, _L) + g * _L
        kk, ii = _sorted_desc(kk, ii)
        ks.append(kk)
        js.append(ii)
    k01, i01 = _merge_top(ks[0], js[0], ks[1], js[1])
    k01, i01 = _sorted_desc(k01, i01)
    k23, i23 = _merge_top(ks[2], js[2], ks[3], js[3])
    k23, i23 = _sorted_desc(k23, i23)
    kf, jf = _merge_top(k01, i01, k23, i23)
    kf, jf = _sorted_desc(kf, jf)
    out_v[r, :] = jf


def _topk_sc(imp):
    R = imp.shape[0]
    rpw = R // _NW
    mesh = plsc.VectorSubcoreMesh(core_axis_name="c", subcore_axis_name="s")

    @functools.partial(
        pl.kernel,
        out_type=jax.ShapeDtypeStruct((R, _TOP_N), jnp.int32),
        mesh=mesh,
        scratch_types=[
            pltpu.VMEM((rpw, 4 * _L), jnp.float32),
            pltpu.VMEM((rpw, _TOP_N), jnp.int32),
        ],
        compiler_params=pltpu.CompilerParams(needs_layout_passes=False),
    )
    def run(imp_hbm, out_hbm, imp_v, out_v):
        wid = lax.axis_index("s") * _NC + lax.axis_index("c")
        base = wid * rpw
        pltpu.sync_copy(imp_hbm.at[pl.ds(base, rpw)], imp_v)

        @plsc.parallel_loop(0, rpw, unroll=4)
        def _(r):
            _row_topk(imp_v, out_v, r)

        pltpu.sync_copy(out_v, out_hbm.at[pl.ds(base, rpw)])

    return run(imp)


def _scores_importance_1d(q, compressed_k):
    B, H, N, D = q.shape
    C = compressed_k.shape[2]
    nn = N // _NT
    grid = (B * nn,)
    return pl.pallas_call(
        _tc_body,
        grid=grid,
        in_specs=[
            pl.BlockSpec((1, H, _NT, D), lambda i: (i // nn, 0, i % nn, 0)),
            pl.BlockSpec((1, H, C, D), lambda i: (i // nn, 0, 0, 0)),
        ],
        out_specs=[
            pl.BlockSpec((1, H, C, _NT), lambda i: (i // nn, 0, 0, i % nn)),
            pl.BlockSpec((1, _NT, C), lambda i: (i // nn, i % nn, 0)),
        ],
        out_shape=[
            jax.ShapeDtypeStruct((B, H, C, N), jnp.float32),
            jax.ShapeDtypeStruct((B, N, C), jnp.float32),
        ],
    )(q, compressed_k)


def kernel(q, compressed_k, k, v):
    del k, v
    B, H, N, D = q.shape
    scores_t, imp = _scores_importance_1d(q, compressed_k)
    scores = jnp.swapaxes(scores_t, 2, 3)
    top = _topk_sc(imp.reshape(B * N, -1)).reshape(B, N, _TOP_N)
    return (top, scores)


# speedup vs baseline: 1.8314x; 1.8314x over previous
import functools

import jax
import jax.numpy as jnp
from jax import lax
from jax.experimental import pallas as pl
from jax.experimental.pallas import tpu as pltpu
from jax.experimental.pallas import tpu_sc as plsc

_TOP_N = 16
_NT = 1024
_L = 16
_NC, _NS = 2, 16
_NW = _NC * _NS


def _tc_body(q_ref, ck_ref, scores_ref, imp_ref):
    nh = q_ref.shape[1]
    c = ck_ref.shape[2]
    d = q_ref.shape[3]
    scale = d ** -0.5
    eye = jnp.eye(c, dtype=jnp.float32)
    for h in range(nh):
        qh = q_ref[0, h]
        ckh = ck_ref[0, h]
        s = jnp.dot(qh, ckh.T, preferred_element_type=jnp.float32) * scale
        m = jnp.max(s, axis=-1, keepdims=True)
        e = jnp.exp(s - m)
        p = e / jnp.sum(e, axis=-1, keepdims=True)
        scores_ref[0, h] = jax.lax.dot_general(
            eye, p, (((1,), (1,)), ((), ())),
            preferred_element_type=jnp.float32)
        if h == 0:
            imp_ref[0] = p
        else:
            imp_ref[0] += p


def _tc_body_alias(q_ref, ck_ref, prev_ref, scores_ref, imp_ref):
    del prev_ref
    _tc_body(q_ref, ck_ref, scores_ref, imp_ref)


def _scores_importance_b(q, compressed_k, b, scores_prev):
    B, H, N, D = q.shape
    C = compressed_k.shape[2]
    grid = (N // _NT,)
    out_specs = [
        pl.BlockSpec((1, H, _NT, C), lambda n: (b, 0, n, 0)),
        pl.BlockSpec((1, _NT, C), lambda n: (0, n, 0)),
    ]
    out_shape = [
        jax.ShapeDtypeStruct((B, H, N, C), jnp.float32),
        jax.ShapeDtypeStruct((1, N, C), jnp.float32),
    ]
    in_specs = [
        pl.BlockSpec((1, H, _NT, D), lambda n: (b, 0, n, 0)),
        pl.BlockSpec((1, H, C, D), lambda n: (b, 0, 0, 0)),
    ]
    if scores_prev is None:
        return pl.pallas_call(
            _tc_body,
            grid=grid,
            in_specs=in_specs,
            out_specs=out_specs,
            out_shape=out_shape,
        )(q, compressed_k)
    return pl.pallas_call(
        _tc_body_alias,
        grid=grid,
        in_specs=in_specs + [pl.BlockSpec(memory_space=pl.ANY)],
        out_specs=out_specs,
        out_shape=out_shape,
        input_output_aliases={2: 0},
    )(q, compressed_k, scores_prev)


def _lane_gather(x, idx):
    return x.at[idx].get(mode="promise_in_bounds")


def _tie_fix(k, i):
    lane = lax.iota(jnp.int32, _L)
    up = jnp.minimum(lane + 1, _L - 1)
    dn = jnp.maximum(lane - 1, 0)
    kn = _lane_gather(k, up)
    for parity in (0, 1):
        inn = _lane_gather(i, up)
        swap = (k == kn) & (i > inn) & ((lane & 1) == parity)
        sw = swap.astype(jnp.int32)
        swd = _lane_gather(sw, dn) > 0
        idn = _lane_gather(i, dn)
        i = jnp.where(swap, inn, jnp.where(swd, idn, i))
    return i


def _sorted_desc(k, i):
    k2, i2 = plsc.sort_key_val(k, i, descending=True)
    return k2, _tie_fix(k2, i2)


def _merge_top(ka, ia, kb, ib):
    kbr = lax.rev(kb, (0,))
    ibr = lax.rev(ib, (0,))
    ta = (ka > kbr) | ((ka == kbr) & (ia < ibr))
    return jnp.where(ta, ka, kbr), jnp.where(ta, ia, ibr)


def _row_topk(imp_v, out_v, r):
    ks, js = [], []
    for g in range(4):
        kk = imp_v[r, pl.ds(g * _L, _L)]
        ii = lax.iota(jnp.int32, _L) + g * _L
        kk, ii = _sorted_desc(kk, ii)
        ks.append(kk)
        js.append(ii)
    k01, i01 = _merge_top(ks[0], js[0], ks[1], js[1])
    k01, i01 = _sorted_desc(k01, i01)
    k23, i23 = _merge_top(ks[2], js[2], ks[3], js[3])
    k23, i23 = _sorted_desc(k23, i23)
    kf, jf = _merge_top(k01, i01, k23, i23)
    kf, jf = _sorted_desc(kf, jf)
    out_v[r, :] = jf


def _topk_sc(imp):
    R = imp.shape[0]
    rpw = R // _NW
    mesh = plsc.VectorSubcoreMesh(core_axis_name="c", subcore_axis_name="s")

    @functools.partial(
        pl.kernel,
        out_type=jax.ShapeDtypeStruct((R, _TOP_N), jnp.int32),
        mesh=mesh,
        scratch_types=[
            pltpu.VMEM((rpw, 4 * _L), jnp.float32),
            pltpu.VMEM((rpw, _TOP_N), jnp.int32),
        ],
        compiler_params=pltpu.CompilerParams(needs_layout_passes=False),
    )
    def run(imp_hbm, out_hbm, imp_v, out_v):
        wid = lax.axis_index("s") * _NC + lax.axis_index("c")
        base = wid * rpw
        pltpu.sync_copy(imp_hbm.at[pl.ds(base, rpw)], imp_v)

        @plsc.parallel_loop(0, rpw, unroll=4)
        def _(r):
            _row_topk(imp_v, out_v, r)

        pltpu.sync_copy(out_v, out_hbm.at[pl.ds(base, rpw)])

    return run(imp)


def _scores_importance_1d(q, compressed_k):
    B, H, N, D = q.shape
    C = compressed_k.shape[2]
    nn = N // _NT
    grid = (B * nn,)
    return pl.pallas_call(
        _tc_body,
        grid=grid,
        in_specs=[
            pl.BlockSpec((1, H, _NT, D), lambda i: (i // nn, 0, i % nn, 0)),
            pl.BlockSpec((1, H, C, D), lambda i: (i // nn, 0, 0, 0)),
        ],
        out_specs=[
            pl.BlockSpec((1, H, C, _NT), lambda i: (i // nn, 0, 0, i % nn)),
            pl.BlockSpec((1, _NT, C), lambda i: (i // nn, i % nn, 0)),
        ],
        out_shape=[
            jax.ShapeDtypeStruct((B, H, C, N), jnp.float32),
            jax.ShapeDtypeStruct((B, N, C), jnp.float32),
        ],
    )(q, compressed_k)


def kernel(q, compressed_k, k, v):
    del k, v
    B, H, N, D = q.shape
    scores_t, imp = _scores_importance_1d(q, compressed_k)
    scores = jnp.swapaxes(scores_t, 2, 3)
    top = _topk_sc(imp.reshape(B * N, -1)).reshape(B, N, _TOP_N)
    return (top, scores)


# speedup vs baseline: 1.8494x; 1.0098x over previous
import functools

import jax
import jax.numpy as jnp
from jax import lax
from jax.experimental import pallas as pl
from jax.experimental.pallas import tpu as pltpu
from jax.experimental.pallas import tpu_sc as plsc

_TOP_N = 16
_NT = 1024
_L = 16
_NC, _NS = 2, 16
_NW = _NC * _NS


def _tc_body(q_ref, ck_ref, scores_ref, imp_ref):
    nh = q_ref.shape[1]
    c = ck_ref.shape[2]
    d = q_ref.shape[3]
    scale = d ** -0.5
    eye = jnp.eye(c, dtype=jnp.float32)
    for h in range(nh):
        qh = q_ref[0, h]
        ckh = ck_ref[0, h]
        s = jnp.dot(qh, ckh.T, preferred_element_type=jnp.float32) * scale
        m = jnp.max(s, axis=-1, keepdims=True)
        e = jnp.exp(s - m)
        p = e / jnp.sum(e, axis=-1, keepdims=True)
        scores_ref[0, h] = jax.lax.dot_general(
            eye, p, (((1,), (1,)), ((), ())),
            preferred_element_type=jnp.float32)
        if h == 0:
            imp_ref[0] = p
        else:
            imp_ref[0] += p


def _tc_body_alias(q_ref, ck_ref, prev_ref, scores_ref, imp_ref):
    del prev_ref
    _tc_body(q_ref, ck_ref, scores_ref, imp_ref)


def _scores_importance_b(q, compressed_k, b, scores_prev):
    B, H, N, D = q.shape
    C = compressed_k.shape[2]
    grid = (N // _NT,)
    out_specs = [
        pl.BlockSpec((1, H, C, _NT), lambda n: (b, 0, 0, n)),
        pl.BlockSpec((1, _NT, C), lambda n: (0, n, 0)),
    ]
    out_shape = [
        jax.ShapeDtypeStruct((B, H, C, N), jnp.float32),
        jax.ShapeDtypeStruct((1, N, C), jnp.float32),
    ]
    in_specs = [
        pl.BlockSpec((1, H, _NT, D), lambda n: (b, 0, n, 0)),
        pl.BlockSpec((1, H, C, D), lambda n: (b, 0, 0, 0)),
    ]
    if scores_prev is None:
        return pl.pallas_call(
            _tc_body,
            grid=grid,
            in_specs=in_specs,
            out_specs=out_specs,
            out_shape=out_shape,
        )(q, compressed_k)
    return pl.pallas_call(
        _tc_body_alias,
        grid=grid,
        in_specs=in_specs + [pl.BlockSpec(memory_space=pl.ANY)],
        out_specs=out_specs,
        out_shape=out_shape,
        input_output_aliases={2: 0},
    )(q, compressed_k, scores_prev)


def _lane_gather(x, idx):
    return x.at[idx].get(mode="promise_in_bounds")


def _tie_fix(k, i):
    lane = lax.iota(jnp.int32, _L)
    up = jnp.minimum(lane + 1, _L - 1)
    dn = jnp.maximum(lane - 1, 0)
    kn = _lane_gather(k, up)
    for parity in (0, 1):
        inn = _lane_gather(i, up)
        swap = (k == kn) & (i > inn) & ((lane & 1) == parity)
        sw = swap.astype(jnp.int32)
        swd = _lane_gather(sw, dn) > 0
        idn = _lane_gather(i, dn)
        i = jnp.where(swap, inn, jnp.where(swd, idn, i))
    return i


def _sorted_desc(k, i):
    k2, i2 = plsc.sort_key_val(k, i, descending=True)
    return k2, _tie_fix(k2, i2)


def _merge_top(ka, ia, kb, ib):
    kbr = lax.rev(kb, (0,))
    ibr = lax.rev(ib, (0,))
    ta = (ka > kbr) | ((ka == kbr) & (ia < ibr))
    return jnp.where(ta, ka, kbr), jnp.where(ta, ia, ibr)


def _row_topk(imp_v, out_v, r):
    ks, js = [], []
    for g in range(4):
        kk = imp_v[r, pl.ds(g * _L, _L)]
        ii = lax.iota(jnp.int32, _L) + g * _L
        kk, ii = _sorted_desc(kk, ii)
        ks.append(kk)
        js.append(ii)
    k01, i01 = _merge_top(ks[0], js[0], ks[1], js[1])
    k01, i01 = _sorted_desc(k01, i01)
    k23, i23 = _merge_top(ks[2], js[2], ks[3], js[3])
    k23, i23 = _sorted_desc(k23, i23)
    kf, jf = _merge_top(k01, i01, k23, i23)
    kf, jf = _sorted_desc(kf, jf)
    out_v[r, :] = jf


def _topk_sc(imp):
    R = imp.shape[0]
    rpw = R // _NW
    mesh = plsc.VectorSubcoreMesh(core_axis_name="c", subcore_axis_name="s")

    @functools.partial(
        pl.kernel,
        out_type=jax.ShapeDtypeStruct((R, _TOP_N), jnp.int32),
        mesh=mesh,
        scratch_types=[
            pltpu.VMEM((rpw, 4 * _L), jnp.float32),
            pltpu.VMEM((rpw, _TOP_N), jnp.int32),
        ],
        compiler_params=pltpu.CompilerParams(needs_layout_passes=False),
    )
    def run(imp_hbm, out_hbm, imp_v, out_v):
        wid = lax.axis_index("s") * _NC + lax.axis_index("c")
        base = wid * rpw
        pltpu.sync_copy(imp_hbm.at[pl.ds(base, rpw)], imp_v)

        @plsc.parallel_loop(0, rpw, unroll=4)
        def _(r):
            _row_topk(imp_v, out_v, r)

        pltpu.sync_copy(out_v, out_hbm.at[pl.ds(base, rpw)])

    return run(imp)


def _scores_importance_1d(q, compressed_k):
    B, H, N, D = q.shape
    C = compressed_k.shape[2]
    nn = N // _NT
    grid = (B * nn,)
    return pl.pallas_call(
        _tc_body,
        grid=grid,
        in_specs=[
            pl.BlockSpec((1, H, _NT, D), lambda i: (i // nn, 0, i % nn, 0)),
            pl.BlockSpec((1, H, C, D), lambda i: (i // nn, 0, 0, 0)),
        ],
        out_specs=[
            pl.BlockSpec((1, H, C, _NT), lambda i: (i // nn, 0, 0, i % nn)),
            pl.BlockSpec((1, _NT, C), lambda i: (i // nn, i % nn, 0)),
        ],
        out_shape=[
            jax.ShapeDtypeStruct((B, H, C, N), jnp.float32),
            jax.ShapeDtypeStruct((B, N, C), jnp.float32),
        ],
    )(q, compressed_k)


def kernel(q, compressed_k, k, v):
    del k, v
    B, H, N, D = q.shape
    scores_t = None
    tops = []
    for b in range(B):
        scores_t, imp_b = _scores_importance_b(q, compressed_k, b, scores_t)
        tops.append(_topk_sc(imp_b.reshape(N, -1)))
    scores = jnp.swapaxes(scores_t, 2, 3)
    top = jnp.concatenate([t[None] for t in tops], axis=0)
    return (top, scores)
